# Initial kernel scaffold; baseline (speedup 1.0000x reference)
import jax
import jax.numpy as jnp
from jax.experimental import pallas as pl


def kernel(item_inputs, table):
    raise NotImplementedError("write your pallas kernel here")


# speedup vs baseline: 1.1105x; 1.1105x over previous
import functools

import jax
import jax.numpy as jnp
from jax import lax
from jax.experimental import pallas as pl
from jax.experimental.pallas import tpu as pltpu
from jax.experimental.pallas import tpu_sc as plsc

BATCH = 16384
HIST = 50
EMBED_DIM = 32
TOTAL = BATCH * HIST

NUM_CORES = 2
NUM_SUBCORES = 16
NUM_WORKERS = NUM_CORES * NUM_SUBCORES
B_PER_W = TOTAL // NUM_WORKERS
CHUNK = 3200
NUM_CHUNKS = B_PER_W // CHUNK

_mesh = plsc.VectorSubcoreMesh(core_axis_name="c", subcore_axis_name="s")


@functools.partial(
    pl.kernel,
    mesh=_mesh,
    out_type=jax.ShapeDtypeStruct((TOTAL, EMBED_DIM), jnp.float32),
    scratch_types=[
        pltpu.VMEM((CHUNK,), jnp.int32),
        pltpu.VMEM((CHUNK, EMBED_DIM), jnp.float32),
        pltpu.SemaphoreType.DMA,
    ],
    compiler_params=pltpu.CompilerParams(use_tc_tiling_on_sc=False),
)
def _gather_kernel(idx_hbm, table_hbm, out_hbm, idx_v, rows_v, sem):
    wid = lax.axis_index("s") * NUM_CORES + lax.axis_index("c")
    base = wid * B_PER_W

    def body(i, _):
        off = base + i * CHUNK
        pltpu.sync_copy(idx_hbm.at[pl.ds(off, CHUNK)], idx_v)
        pltpu.async_copy(table_hbm.at[idx_v], rows_v, sem).wait()
        pltpu.sync_copy(rows_v, out_hbm.at[pl.ds(off, CHUNK)])
        return 0

    lax.fori_loop(0, NUM_CHUNKS, body, 0)


def kernel(item_inputs, table):
    flat_idx = item_inputs.reshape(TOTAL).astype(jnp.int32)
    out = _gather_kernel(flat_idx, table)
    return out.reshape(BATCH, HIST, EMBED_DIM)


# speedup vs baseline: 1.1140x; 1.0031x over previous
import functools

import jax
import jax.numpy as jnp
from jax import lax
from jax.experimental import pallas as pl
from jax.experimental.pallas import tpu as pltpu
from jax.experimental.pallas import tpu_sc as plsc

BATCH = 16384
HIST = 50
EMBED_DIM = 32
TOTAL = BATCH * HIST

NUM_CORES = 2
NUM_SUBCORES = 16
NUM_WORKERS = NUM_CORES * NUM_SUBCORES
B_PER_W = TOTAL // NUM_WORKERS
CHUNK = 1600
NUM_CHUNKS = B_PER_W // CHUNK

_mesh = plsc.VectorSubcoreMesh(core_axis_name="c", subcore_axis_name="s")


@functools.partial(
    pl.kernel,
    mesh=_mesh,
    out_type=jax.ShapeDtypeStruct((TOTAL, EMBED_DIM), jnp.float32),
    scratch_types=[
        pltpu.VMEM((B_PER_W,), jnp.int32),
        pltpu.VMEM((CHUNK, EMBED_DIM), jnp.float32),
        pltpu.VMEM((CHUNK, EMBED_DIM), jnp.float32),
        pltpu.SemaphoreType.DMA,
        pltpu.SemaphoreType.DMA,
        pltpu.SemaphoreType.DMA,
        pltpu.SemaphoreType.DMA,
    ],
    compiler_params=pltpu.CompilerParams(use_tc_tiling_on_sc=False),
)
def _gather_kernel(idx_hbm, table_hbm, out_hbm, idx_all, rows0, rows1,
                   gs0, gs1, ws0, ws1):
    wid = lax.axis_index("s") * NUM_CORES + lax.axis_index("c")
    base = wid * B_PER_W
    pltpu.sync_copy(idx_hbm.at[pl.ds(base, B_PER_W)], idx_all)

    rows = (rows0, rows1)
    gs = (gs0, gs1)
    ws = (ws0, ws1)

    def fire_gather(i):
        p = i % 2
        idx_slice = idx_all.at[pl.ds(i * CHUNK, CHUNK)]
        return pltpu.async_copy(table_hbm.at[idx_slice], rows[p], gs[p])

    gd = [None] * NUM_CHUNKS
    wd = [None] * NUM_CHUNKS
    gd[0] = fire_gather(0)
    gd[1] = fire_gather(1)
    for i in range(NUM_CHUNKS):
        p = i % 2
        gd[i].wait()
        wd[i] = pltpu.async_copy(
            rows[p], out_hbm.at[pl.ds(base + i * CHUNK, CHUNK)], ws[p])
        if i + 2 < NUM_CHUNKS:
            wd[i].wait()
            gd[i + 2] = fire_gather(i + 2)
    wd[NUM_CHUNKS - 2].wait()
    wd[NUM_CHUNKS - 1].wait()


def kernel(item_inputs, table):
    flat_idx = item_inputs.reshape(TOTAL).astype(jnp.int32)
    out = _gather_kernel(flat_idx, table)
    return out.reshape(BATCH, HIST, EMBED_DIM)


# speedup vs baseline: 1.9424x; 1.7437x over previous
import functools

import jax
import jax.numpy as jnp
from jax import lax
from jax.experimental import pallas as pl
from jax.experimental.pallas import tpu as pltpu
from jax.experimental.pallas import tpu_sc as plsc

BATCH = 16384
HIST = 50
EMBED_DIM = 32
TOTAL = BATCH * HIST

NUM_CORES = 2
NUM_SUBCORES = 16
NUM_WORKERS = NUM_CORES * NUM_SUBCORES
B_PER_W = TOTAL // NUM_WORKERS
CHUNK = 1600
NUM_CHUNKS = B_PER_W // CHUNK

_mesh = plsc.VectorSubcoreMesh(core_axis_name="c", subcore_axis_name="s")


@functools.partial(
    pl.kernel,
    mesh=_mesh,
    out_type=jax.ShapeDtypeStruct((TOTAL, EMBED_DIM), jnp.float32),
    scratch_types=[
        pltpu.VMEM((B_PER_W,), jnp.int32),
        pltpu.VMEM((CHUNK, EMBED_DIM), jnp.float32),
        pltpu.VMEM((CHUNK, EMBED_DIM), jnp.float32),
        pltpu.SemaphoreType.DMA,
        pltpu.SemaphoreType.DMA,
        pltpu.SemaphoreType.DMA,
        pltpu.SemaphoreType.DMA,
    ],
    compiler_params=pltpu.CompilerParams(use_tc_tiling_on_sc=False),
)
def _gather_kernel(idx_hbm, table_hbm, out_hbm, idx_all, rows0, rows1,
                   gs0, gs1, ws0, ws1):
    wid = lax.axis_index("s") * NUM_CORES + lax.axis_index("c")
    base = wid * B_PER_W
    pltpu.sync_copy(idx_hbm.at[pl.ds(base, B_PER_W)], idx_all)

    rows = (rows0, rows1)
    gs = (gs0, gs1)
    ws = (ws0, ws1)

    def fire_gather(i):
        p = i % 2
        idx_slice = idx_all.at[pl.ds(i * CHUNK, CHUNK)]
        return pltpu.async_copy(table_hbm.at[idx_slice], rows[p], gs[p])

    gd = [None] * NUM_CHUNKS
    wd = [None] * NUM_CHUNKS
    gd[0] = fire_gather(0)
    gd[1] = fire_gather(1)
    for i in range(NUM_CHUNKS):
        p = i % 2
        gd[i].wait()
        wd[i] = pltpu.async_copy(
            rows[p], out_hbm.at[pl.ds(base + i * CHUNK, CHUNK)], ws[p])
        if i + 2 < NUM_CHUNKS:
            wd[i].wait()
            gd[i + 2] = fire_gather(i + 2)
    wd[NUM_CHUNKS - 2].wait()
    wd[NUM_CHUNKS - 1].wait()


def kernel(item_inputs, table):
    flat_idx = item_inputs.astype(jnp.int32).T.reshape(TOTAL)
    out = _gather_kernel(flat_idx, table)
    return out.reshape(HIST, BATCH, EMBED_DIM).transpose(1, 0, 2)
